# baseline (speedup 1.0000x reference)
import jax
import jax.numpy as jnp
from jax import lax
from jax.experimental import pallas as pl
from jax.experimental.pallas import tpu as pltpu
from jax.experimental.pallas import tpu_sc as plsc

_N = 100000
_DIN = 128
_DOUT = 32
_B = 10000
_DEG = 16
_ALPHA = 0.2

_NC, _NS = 2, 16
_NW = _NC * _NS
_LANES = 128

_SELF_IR = 3
_SELF_PW = _SELF_IR * _LANES
_SELF_PAD = _NW * _SELF_PW
_NB_IR = 40
_NB_PW = _NB_IR * _LANES
_NB_PAD = _NW * _NB_PW
_CH_IR = 10
_CHUNK = _CH_IR * _LANES


def _leaky(x):
    return jnp.where(x >= 0, x, _ALPHA * x)


_PROJ_ROWS = 2000


def _proj_body(f_ref, wb_ref, wu_ref, hb_ref, hu_ref):
    f = f_ref[...]
    hb_ref[...] = jnp.dot(f, wb_ref[...], preferred_element_type=jnp.float32)
    hu_ref[...] = jnp.dot(f, wu_ref[...], preferred_element_type=jnp.float32)


def _project(feat, wb, wu):
    return pl.pallas_call(
        _proj_body,
        grid=(_N // _PROJ_ROWS,),
        in_specs=[
            pl.BlockSpec((_PROJ_ROWS, _DIN), lambda i: (i, 0)),
            pl.BlockSpec((_DIN, _DOUT), lambda i: (0, 0)),
            pl.BlockSpec((_DIN, _DOUT), lambda i: (0, 0)),
        ],
        out_specs=[
            pl.BlockSpec((_PROJ_ROWS, _DOUT), lambda i: (i, 0)),
            pl.BlockSpec((_PROJ_ROWS, _DOUT), lambda i: (i, 0)),
        ],
        out_shape=[
            jax.ShapeDtypeStruct((_N, _DOUT), jnp.float32),
            jax.ShapeDtypeStruct((_N, _DOUT), jnp.float32),
        ],
    )(feat, wb, wu)


def _fire(tbl, idx, i_off, rows, sem):
    return [
        pltpu.async_copy(tbl.at[idx.at[pl.ds(i_off + j * _LANES, _LANES)]],
                         rows.at[pl.ds(j * _LANES, _LANES)], sem)
        for j in range(_CH_IR)
    ]


_IB_ROWS = 1280


def _gather_body(hb, hu, iself, ipos, ineg,
                 sb, su, nbp, nbn,
                 idx_s, idx_n, ibuf, rows_a, rows_b, sem_a, sem_b):
    wid = lax.axis_index("s") * _NC + lax.axis_index("c")
    s_base = pl.multiple_of(wid * _SELF_PW, _SELF_PW)
    n_base = pl.multiple_of(wid * _NB_PW, _NB_PW)
    j_col = wid // _NC
    r_base = pl.multiple_of((wid % _NC) * _NB_PW, _NB_PW)
    lanes16 = lax.iota(jnp.int32, 16)
    col16 = jnp.full((16,), 0, jnp.int32) + j_col

    pltpu.sync_copy(iself.at[pl.ds(s_base, _SELF_PW)], idx_s)
    da = [pltpu.async_copy(hb.at[idx_s.at[pl.ds(j * _LANES, _LANES)]],
                           rows_a.at[pl.ds(j * _LANES, _LANES)], sem_a)
          for j in range(_SELF_IR)]
    db = [pltpu.async_copy(hu.at[idx_s.at[pl.ds(j * _LANES, _LANES)]],
                           rows_b.at[pl.ds(j * _LANES, _LANES)], sem_b)
          for j in range(_SELF_IR)]
    for d in da:
        d.wait()
    pltpu.sync_copy(rows_a.at[pl.ds(0, _SELF_PW)],
                    sb.at[pl.ds(s_base, _SELF_PW)])
    for d in db:
        d.wait()
    pltpu.sync_copy(rows_b.at[pl.ds(0, _SELF_PW)],
                    su.at[pl.ds(s_base, _SELF_PW)])

    for itab, tbl, out in ((ipos, hb, nbp), (ineg, hu, nbn)):
        def extract(c, carry):
            c_off = pl.multiple_of(c * _IB_ROWS, _IB_ROWS)
            pltpu.sync_copy(itab.at[pl.ds(r_base + c_off, _IB_ROWS)], ibuf)

            def grp(g, carry2):
                v = plsc.load_gather(ibuf, [g * 16 + lanes16, col16])
                idx_n[pl.ds(c_off + g * 16, 16)] = v
                return carry2

            lax.fori_loop(0, _IB_ROWS // 16, grp, 0)
            return carry

        lax.fori_loop(0, _NB_PW // _IB_ROWS, extract, 0)

        def pair(p, carry):
            off_a = pl.multiple_of(p * (2 * _CHUNK), 2 * _CHUNK)
            off_b = off_a + _CHUNK
            da = _fire(tbl, idx_n, off_a, rows_a, sem_a)
            db = _fire(tbl, idx_n, off_b, rows_b, sem_b)
            for d in da:
                d.wait()
            pltpu.sync_copy(rows_a, out.at[pl.ds(n_base + off_a, _CHUNK)])
            for d in db:
                d.wait()
            pltpu.sync_copy(rows_b, out.at[pl.ds(n_base + off_b, _CHUNK)])
            return carry

        lax.fori_loop(0, _NB_PW // (2 * _CHUNK), pair, 0)


def _gather(hb, hu, iself, ipos, ineg):
    mesh = plsc.VectorSubcoreMesh(core_axis_name="c", subcore_axis_name="s")
    f = pl.kernel(
        _gather_body,
        mesh=mesh,
        out_type=[
            jax.ShapeDtypeStruct((_SELF_PAD, _DOUT), jnp.float32),
            jax.ShapeDtypeStruct((_SELF_PAD, _DOUT), jnp.float32),
            jax.ShapeDtypeStruct((_NB_PAD, _DOUT), jnp.float32),
            jax.ShapeDtypeStruct((_NB_PAD, _DOUT), jnp.float32),
        ],
        scratch_types=[
            pltpu.VMEM((_SELF_PW,), jnp.int32),
            pltpu.VMEM((_NB_PW,), jnp.int32),
            pltpu.VMEM((_IB_ROWS, _DEG), jnp.int32),
            pltpu.VMEM((_CHUNK, _DOUT), jnp.float32),
            pltpu.VMEM((_CHUNK, _DOUT), jnp.float32),
            pltpu.SemaphoreType.DMA,
            pltpu.SemaphoreType.DMA,
        ],
        compiler_params=pltpu.CompilerParams(use_tc_tiling_on_sc=False,
                                             needs_layout_passes=False),
    )
    return f(hb, hu, iself, ipos, ineg)


_SEG = _LANES // _DOUT
_BPAD = 10240
_TB = _BPAD // _SEG


def _attn_body(hsb, hsu, npos, nneg, a1b, a2b, a1u, a2u,
               xb, xu, rtb, rtu, dnb, dnu):
    j = pl.program_id(0)
    seg = (lax.broadcasted_iota(jnp.int32, (_SEG, _LANES), 1) // _DOUT
           == lax.broadcasted_iota(jnp.int32, (_SEG, _LANES), 0))
    expand = seg.astype(jnp.float32)

    @pl.when(j == 0)
    def _init():
        for hs, a1, a2, rt, dn, x in ((hsb, a1b, a2b, rtb, dnb, xb),
                                      (hsu, a1u, a2u, rtu, dnu, xu)):
            h = hs[...]
            r = jnp.dot(h, a1[...], preferred_element_type=jnp.float32)
            e = jnp.exp(-_leaky(
                r + jnp.dot(h, a2[...], preferred_element_type=jnp.float32)))
            rt[...] = r
            dn[...] = e
            x[...] = jnp.dot(e, expand,
                             preferred_element_type=jnp.float32) * h

    for nb, a2, rt, dn, x in ((npos, a2b, rtb, dnb, xb),
                              (nneg, a2u, rtu, dnu, xu)):
        h = nb[...]
        e = jnp.exp(-_leaky(
            rt[...] + jnp.dot(h, a2[...],
                              preferred_element_type=jnp.float32)))
        dn[...] += e
        x[...] += jnp.dot(e, expand, preferred_element_type=jnp.float32) * h

    @pl.when(j == _DEG - 1)
    def _final():
        for x, dn in ((xb, dnb), (xu, dnu)):
            d = jnp.dot(dn[...] + 1e-16, expand,
                        preferred_element_type=jnp.float32)
            v = x[...] / d
            x[...] = jnp.where(v > 0, v, jnp.exp(v) - 1.0)


def _attention(hsb, hsu, npos, nneg, a1b, a2b, a1u, a2u):
    return pl.pallas_call(
        _attn_body,
        grid=(_DEG,),
        in_specs=[
            pl.BlockSpec((_TB, _LANES), lambda j: (0, 0)),
            pl.BlockSpec((_TB, _LANES), lambda j: (0, 0)),
            pl.BlockSpec((_TB, _LANES), lambda j: (j, 0)),
            pl.BlockSpec((_TB, _LANES), lambda j: (j, 0)),
            pl.BlockSpec((_DIN, _SEG), lambda j: (0, 0)),
            pl.BlockSpec((_DIN, _SEG), lambda j: (0, 0)),
            pl.BlockSpec((_DIN, _SEG), lambda j: (0, 0)),
            pl.BlockSpec((_DIN, _SEG), lambda j: (0, 0)),
        ],
        out_specs=[
            pl.BlockSpec((_TB, _LANES), lambda j: (0, 0)),
            pl.BlockSpec((_TB, _LANES), lambda j: (0, 0)),
        ],
        out_shape=[
            jax.ShapeDtypeStruct((_TB, _LANES), jnp.float32),
            jax.ShapeDtypeStruct((_TB, _LANES), jnp.float32),
        ],
        scratch_shapes=[pltpu.VMEM((_TB, _SEG), jnp.float32)] * 4,
    )(hsb, hsu, npos, nneg, a1b, a2b, a1u, a2u)


def kernel(nodes, neigh_pos, neigh_neg, feat_table,
           W_bal, a_bal, W_unbal, a_unbal):
    hb, hu = _project(feat_table, W_bal, W_unbal)

    zs = jnp.zeros(_SELF_PAD - _B, jnp.int32)
    zr = jnp.zeros((_BPAD - _B, _DEG), jnp.int32)
    iself = jnp.concatenate([nodes, zs])
    ipos = jnp.concatenate([neigh_pos, zr], axis=0)
    ineg = jnp.concatenate([neigh_neg, zr], axis=0)

    sb, su, nbp, nbn = _gather(hb, hu, iself, ipos, ineg)

    hs2b = sb.reshape(-1, _LANES)[:_TB]
    hs2u = su.reshape(-1, _LANES)[:_TB]
    np2 = nbp.reshape(-1, _LANES)
    nn2 = nbn.reshape(-1, _LANES)

    eye = jnp.eye(_SEG, dtype=jnp.float32)
    a1b = jnp.kron(eye, a_bal[0, :_DOUT][:, None])
    a2b = jnp.kron(eye, a_bal[0, _DOUT:][:, None])
    a1u = jnp.kron(eye, a_unbal[0, :_DOUT][:, None])
    a2u = jnp.kron(eye, a_unbal[0, _DOUT:][:, None])

    xb2, xu2 = _attention(hs2b, hs2u, np2, nn2, a1b, a2b, a1u, a2u)
    return (xb2.reshape(-1, _DOUT)[:_B], xu2.reshape(-1, _DOUT)[:_B])


# speedup vs baseline: 1.0031x; 1.0031x over previous
import jax
import jax.numpy as jnp
from jax import lax
from jax.experimental import pallas as pl
from jax.experimental.pallas import tpu as pltpu
from jax.experimental.pallas import tpu_sc as plsc

_N = 100000
_DIN = 128
_DOUT = 32
_B = 10000
_DEG = 16
_ALPHA = 0.2

_NC, _NS = 2, 16
_NW = _NC * _NS
_LANES = 128

_SELF_IR = 3
_SELF_PW = _SELF_IR * _LANES
_SELF_PAD = _NW * _SELF_PW
_NB_IR = 40
_NB_PW = _NB_IR * _LANES
_NB_PAD = _NW * _NB_PW
_CH_IR = 10
_CHUNK = _CH_IR * _LANES


def _leaky(x):
    return jnp.where(x >= 0, x, _ALPHA * x)


_PROJ_ROWS = 2000


def _proj_body(f_ref, wb_ref, wu_ref, hb_ref, hu_ref):
    f = f_ref[...]
    hb_ref[...] = jnp.dot(f, wb_ref[...], preferred_element_type=jnp.float32)
    hu_ref[...] = jnp.dot(f, wu_ref[...], preferred_element_type=jnp.float32)


def _project(feat, wb, wu):
    return pl.pallas_call(
        _proj_body,
        grid=(_N // _PROJ_ROWS,),
        in_specs=[
            pl.BlockSpec((_PROJ_ROWS, _DIN), lambda i: (i, 0)),
            pl.BlockSpec((_DIN, _DOUT), lambda i: (0, 0)),
            pl.BlockSpec((_DIN, _DOUT), lambda i: (0, 0)),
        ],
        out_specs=[
            pl.BlockSpec((_PROJ_ROWS, _DOUT), lambda i: (i, 0)),
            pl.BlockSpec((_PROJ_ROWS, _DOUT), lambda i: (i, 0)),
        ],
        out_shape=[
            jax.ShapeDtypeStruct((_N, _DOUT), jnp.float32),
            jax.ShapeDtypeStruct((_N, _DOUT), jnp.float32),
        ],
    )(feat, wb, wu)


def _fire(tbl, idx, i_off, rows, sem):
    return [
        pltpu.async_copy(tbl.at[idx.at[pl.ds(i_off + j * _LANES, _LANES)]],
                         rows.at[pl.ds(j * _LANES, _LANES)], sem)
        for j in range(_CH_IR)
    ]


def _gather_body(hb, hu, iself, ipos, ineg,
                 sb, su, nbp, nbn,
                 idx_s, idx_n, rows_a, rows_b, sem_a, sem_b):
    wid = lax.axis_index("s") * _NC + lax.axis_index("c")
    s_base = pl.multiple_of(wid * _SELF_PW, _SELF_PW)
    n_base = pl.multiple_of(wid * _NB_PW, _NB_PW)
    j_col = wid // _NC
    r_base = pl.multiple_of((wid % _NC) * _NB_PW, _NB_PW)

    pltpu.sync_copy(iself.at[pl.ds(s_base, _SELF_PW)], idx_s)
    da = [pltpu.async_copy(hb.at[idx_s.at[pl.ds(j * _LANES, _LANES)]],
                           rows_a.at[pl.ds(j * _LANES, _LANES)], sem_a)
          for j in range(_SELF_IR)]
    db = [pltpu.async_copy(hu.at[idx_s.at[pl.ds(j * _LANES, _LANES)]],
                           rows_b.at[pl.ds(j * _LANES, _LANES)], sem_b)
          for j in range(_SELF_IR)]
    for d in da:
        d.wait()
    pltpu.sync_copy(rows_a.at[pl.ds(0, _SELF_PW)],
                    sb.at[pl.ds(s_base, _SELF_PW)])
    for d in db:
        d.wait()
    pltpu.sync_copy(rows_b.at[pl.ds(0, _SELF_PW)],
                    su.at[pl.ds(s_base, _SELF_PW)])

    for itab, tbl, out in ((ipos, hb, nbp), (ineg, hu, nbn)):
        pltpu.sync_copy(itab.at[j_col, pl.ds(r_base, _NB_PW)], idx_n)

        def pair(p, carry):
            off_a = pl.multiple_of(p * (2 * _CHUNK), 2 * _CHUNK)
            off_b = off_a + _CHUNK
            da = _fire(tbl, idx_n, off_a, rows_a, sem_a)
            db = _fire(tbl, idx_n, off_b, rows_b, sem_b)
            for d in da:
                d.wait()
            pltpu.sync_copy(rows_a, out.at[pl.ds(n_base + off_a, _CHUNK)])
            for d in db:
                d.wait()
            pltpu.sync_copy(rows_b, out.at[pl.ds(n_base + off_b, _CHUNK)])
            return carry

        lax.fori_loop(0, _NB_PW // (2 * _CHUNK), pair, 0)


def _gather(hb, hu, iself, ipos, ineg):
    mesh = plsc.VectorSubcoreMesh(core_axis_name="c", subcore_axis_name="s")
    f = pl.kernel(
        _gather_body,
        mesh=mesh,
        out_type=[
            jax.ShapeDtypeStruct((_SELF_PAD, _DOUT), jnp.float32),
            jax.ShapeDtypeStruct((_SELF_PAD, _DOUT), jnp.float32),
            jax.ShapeDtypeStruct((_NB_PAD, _DOUT), jnp.float32),
            jax.ShapeDtypeStruct((_NB_PAD, _DOUT), jnp.float32),
        ],
        scratch_types=[
            pltpu.VMEM((_SELF_PW,), jnp.int32),
            pltpu.VMEM((_NB_PW,), jnp.int32),
            pltpu.VMEM((_CHUNK, _DOUT), jnp.float32),
            pltpu.VMEM((_CHUNK, _DOUT), jnp.float32),
            pltpu.SemaphoreType.DMA,
            pltpu.SemaphoreType.DMA,
        ],
        compiler_params=pltpu.CompilerParams(use_tc_tiling_on_sc=False,
                                             needs_layout_passes=False),
    )
    return f(hb, hu, iself, ipos, ineg)


_BPAD_T = 10240


def _tr_body(p_ref, n_ref, pt_ref, nt_ref):
    pt_ref[...] = jnp.transpose(p_ref[...])
    nt_ref[...] = jnp.transpose(n_ref[...])


def _transpose_idx(ipos, ineg):
    return pl.pallas_call(
        _tr_body,
        in_specs=[pl.BlockSpec((_BPAD_T, _DEG), lambda: (0, 0))] * 2,
        out_specs=[pl.BlockSpec((_DEG, _BPAD_T), lambda: (0, 0))] * 2,
        out_shape=[jax.ShapeDtypeStruct((_DEG, _BPAD_T), jnp.int32)] * 2,
    )(ipos, ineg)


_SEG = _LANES // _DOUT
_BPAD = 10240
_TB = _BPAD // _SEG


def _attn_body(hsb, hsu, npos, nneg, a1b, a2b, a1u, a2u,
               xb, xu, rtb, rtu, dnb, dnu):
    j = pl.program_id(0)
    seg = (lax.broadcasted_iota(jnp.int32, (_SEG, _LANES), 1) // _DOUT
           == lax.broadcasted_iota(jnp.int32, (_SEG, _LANES), 0))
    expand = seg.astype(jnp.float32)

    @pl.when(j == 0)
    def _init():
        for hs, a1, a2, rt, dn, x in ((hsb, a1b, a2b, rtb, dnb, xb),
                                      (hsu, a1u, a2u, rtu, dnu, xu)):
            h = hs[...]
            r = jnp.dot(h, a1[...], preferred_element_type=jnp.float32)
            e = jnp.exp(-_leaky(
                r + jnp.dot(h, a2[...], preferred_element_type=jnp.float32)))
            rt[...] = r
            dn[...] = e
            x[...] = jnp.dot(e, expand,
                             preferred_element_type=jnp.float32) * h

    for nb, a2, rt, dn, x in ((npos, a2b, rtb, dnb, xb),
                              (nneg, a2u, rtu, dnu, xu)):
        h = nb[...]
        e = jnp.exp(-_leaky(
            rt[...] + jnp.dot(h, a2[...],
                              preferred_element_type=jnp.float32)))
        dn[...] += e
        x[...] += jnp.dot(e, expand, preferred_element_type=jnp.float32) * h

    @pl.when(j == _DEG - 1)
    def _final():
        for x, dn in ((xb, dnb), (xu, dnu)):
            d = jnp.dot(dn[...] + 1e-16, expand,
                        preferred_element_type=jnp.float32)
            v = x[...] / d
            x[...] = jnp.where(v > 0, v, jnp.exp(v) - 1.0)


def _attention(hsb, hsu, npos, nneg, a1b, a2b, a1u, a2u):
    return pl.pallas_call(
        _attn_body,
        grid=(_DEG,),
        in_specs=[
            pl.BlockSpec((_TB, _LANES), lambda j: (0, 0)),
            pl.BlockSpec((_TB, _LANES), lambda j: (0, 0)),
            pl.BlockSpec((_TB, _LANES), lambda j: (j, 0)),
            pl.BlockSpec((_TB, _LANES), lambda j: (j, 0)),
            pl.BlockSpec((_DIN, _SEG), lambda j: (0, 0)),
            pl.BlockSpec((_DIN, _SEG), lambda j: (0, 0)),
            pl.BlockSpec((_DIN, _SEG), lambda j: (0, 0)),
            pl.BlockSpec((_DIN, _SEG), lambda j: (0, 0)),
        ],
        out_specs=[
            pl.BlockSpec((_TB, _LANES), lambda j: (0, 0)),
            pl.BlockSpec((_TB, _LANES), lambda j: (0, 0)),
        ],
        out_shape=[
            jax.ShapeDtypeStruct((_TB, _LANES), jnp.float32),
            jax.ShapeDtypeStruct((_TB, _LANES), jnp.float32),
        ],
        scratch_shapes=[pltpu.VMEM((_TB, _SEG), jnp.float32)] * 4,
    )(hsb, hsu, npos, nneg, a1b, a2b, a1u, a2u)


def kernel(nodes, neigh_pos, neigh_neg, feat_table,
           W_bal, a_bal, W_unbal, a_unbal):
    hb, hu = _project(feat_table, W_bal, W_unbal)

    zs = jnp.zeros(_SELF_PAD - _B, jnp.int32)
    zr = jnp.zeros((_BPAD - _B, _DEG), jnp.int32)
    iself = jnp.concatenate([nodes, zs])
    ipos = jnp.concatenate([neigh_pos, zr], axis=0)
    ineg = jnp.concatenate([neigh_neg, zr], axis=0)
    ipt, int_ = _transpose_idx(ipos, ineg)

    sb, su, nbp, nbn = _gather(hb, hu, iself, ipt, int_)

    hs2b = sb.reshape(-1, _LANES)[:_TB]
    hs2u = su.reshape(-1, _LANES)[:_TB]
    np2 = nbp.reshape(-1, _LANES)
    nn2 = nbn.reshape(-1, _LANES)

    eye = jnp.eye(_SEG, dtype=jnp.float32)
    a1b = jnp.kron(eye, a_bal[0, :_DOUT][:, None])
    a2b = jnp.kron(eye, a_bal[0, _DOUT:][:, None])
    a1u = jnp.kron(eye, a_unbal[0, :_DOUT][:, None])
    a2u = jnp.kron(eye, a_unbal[0, _DOUT:][:, None])

    xb2, xu2 = _attention(hs2b, hs2u, np2, nn2, a1b, a2b, a1u, a2u)
    return (xb2.reshape(-1, _DOUT)[:_B], xu2.reshape(-1, _DOUT)[:_B])


# speedup vs baseline: 1.1148x; 1.1114x over previous
import jax
import jax.numpy as jnp
from jax import lax
from jax.experimental import pallas as pl
from jax.experimental.pallas import tpu as pltpu
from jax.experimental.pallas import tpu_sc as plsc

_N = 100000
_DIN = 128
_DOUT = 32
_B = 10000
_DEG = 16
_ALPHA = 0.2

_NC, _NS = 2, 16
_NW = _NC * _NS
_LANES = 128

_SELF_IR = 3
_SELF_PW = _SELF_IR * _LANES
_SELF_PAD = _NW * _SELF_PW
_NB_IR = 40
_NB_PW = _NB_IR * _LANES
_NB_PAD = _NW * _NB_PW
_CH_IR = 10
_CHUNK = _CH_IR * _LANES


def _leaky(x):
    return jnp.where(x >= 0, x, _ALPHA * x)


_PROJ_ROWS = 2000


def _proj_body(f_ref, wb_ref, wu_ref, h_ref):
    f = f_ref[...]
    hb = jnp.dot(f, wb_ref[...], preferred_element_type=jnp.float32)
    hu = jnp.dot(f, wu_ref[...], preferred_element_type=jnp.float32)
    z = jnp.zeros((_PROJ_ROWS, _DIN - 2 * _DOUT), jnp.float32)
    h_ref[...] = jnp.concatenate([hb, hu, z], axis=1)


def _project(feat, wb, wu):
    return pl.pallas_call(
        _proj_body,
        grid=(_N // _PROJ_ROWS,),
        in_specs=[
            pl.BlockSpec((_PROJ_ROWS, _DIN), lambda i: (i, 0)),
            pl.BlockSpec((_DIN, _DOUT), lambda i: (0, 0)),
            pl.BlockSpec((_DIN, _DOUT), lambda i: (0, 0)),
        ],
        out_specs=pl.BlockSpec((_PROJ_ROWS, _DIN), lambda i: (i, 0)),
        out_shape=jax.ShapeDtypeStruct((_N, _DIN), jnp.float32),
    )(feat, wb, wu)


def _fire(tbl, idx, i_off, rows, sem):
    return [
        pltpu.async_copy(tbl.at[idx.at[pl.ds(i_off + j * _LANES, _LANES)]],
                         rows.at[pl.ds(j * _LANES, _LANES)], sem)
        for j in range(_CH_IR)
    ]


def _gather_body(tbl, isb, isu, ipos, ineg,
                 sb, su, nbp, nbn,
                 idx_s, idx_s2, idx_n, rows_a, rows_b, sem_a, sem_b):
    wid = lax.axis_index("s") * _NC + lax.axis_index("c")
    s_base = pl.multiple_of(wid * _SELF_PW, _SELF_PW)
    n_base = pl.multiple_of(wid * _NB_PW, _NB_PW)
    j_col = wid // _NC
    r_base = pl.multiple_of((wid % _NC) * _NB_PW, _NB_PW)

    pltpu.sync_copy(isb.at[pl.ds(s_base, _SELF_PW)], idx_s)
    pltpu.sync_copy(isu.at[pl.ds(s_base, _SELF_PW)], idx_s2)
    da = [pltpu.async_copy(tbl.at[idx_s.at[pl.ds(j * _LANES, _LANES)]],
                           rows_a.at[pl.ds(j * _LANES, _LANES)], sem_a)
          for j in range(_SELF_IR)]
    db = [pltpu.async_copy(tbl.at[idx_s2.at[pl.ds(j * _LANES, _LANES)]],
                           rows_b.at[pl.ds(j * _LANES, _LANES)], sem_b)
          for j in range(_SELF_IR)]
    for d in da:
        d.wait()
    pltpu.sync_copy(rows_a.at[pl.ds(0, _SELF_PW)],
                    sb.at[pl.ds(s_base, _SELF_PW)])
    for d in db:
        d.wait()
    pltpu.sync_copy(rows_b.at[pl.ds(0, _SELF_PW)],
                    su.at[pl.ds(s_base, _SELF_PW)])

    for itab, out in ((ipos, nbp), (ineg, nbn)):
        pltpu.sync_copy(itab.at[j_col, pl.ds(r_base, _NB_PW)], idx_n)

        def pair(p, carry):
            off_a = pl.multiple_of(p * (2 * _CHUNK), 2 * _CHUNK)
            off_b = off_a + _CHUNK
            da = _fire(tbl, idx_n, off_a, rows_a, sem_a)
            db = _fire(tbl, idx_n, off_b, rows_b, sem_b)
            for d in da:
                d.wait()
            pltpu.sync_copy(rows_a, out.at[pl.ds(n_base + off_a, _CHUNK)])
            for d in db:
                d.wait()
            pltpu.sync_copy(rows_b, out.at[pl.ds(n_base + off_b, _CHUNK)])
            return carry

        lax.fori_loop(0, _NB_PW // (2 * _CHUNK), pair, 0)


def _gather(tbl, isb, isu, ipos, ineg):
    mesh = plsc.VectorSubcoreMesh(core_axis_name="c", subcore_axis_name="s")
    f = pl.kernel(
        _gather_body,
        mesh=mesh,
        out_type=[
            jax.ShapeDtypeStruct((_SELF_PAD, _DOUT), jnp.float32),
            jax.ShapeDtypeStruct((_SELF_PAD, _DOUT), jnp.float32),
            jax.ShapeDtypeStruct((_NB_PAD, _DOUT), jnp.float32),
            jax.ShapeDtypeStruct((_NB_PAD, _DOUT), jnp.float32),
        ],
        scratch_types=[
            pltpu.VMEM((_SELF_PW,), jnp.int32),
            pltpu.VMEM((_SELF_PW,), jnp.int32),
            pltpu.VMEM((_NB_PW,), jnp.int32),
            pltpu.VMEM((_CHUNK, _DOUT), jnp.float32),
            pltpu.VMEM((_CHUNK, _DOUT), jnp.float32),
            pltpu.SemaphoreType.DMA,
            pltpu.SemaphoreType.DMA,
        ],
        compiler_params=pltpu.CompilerParams(use_tc_tiling_on_sc=False,
                                             needs_layout_passes=False),
    )
    return f(tbl, isb, isu, ipos, ineg)


_BPAD_T = 10240


def _tr_body(p_ref, n_ref, pt_ref, nt_ref):
    pt_ref[...] = jnp.transpose(p_ref[...]) * 4
    nt_ref[...] = jnp.transpose(n_ref[...]) * 4 + 1


def _transpose_idx(ipos, ineg):
    return pl.pallas_call(
        _tr_body,
        in_specs=[pl.BlockSpec((_BPAD_T, _DEG), lambda: (0, 0))] * 2,
        out_specs=[pl.BlockSpec((_DEG, _BPAD_T), lambda: (0, 0))] * 2,
        out_shape=[jax.ShapeDtypeStruct((_DEG, _BPAD_T), jnp.int32)] * 2,
    )(ipos, ineg)


_SEG = _LANES // _DOUT
_BPAD = 10240
_TB = _BPAD // _SEG


def _attn_body(hsb, hsu, npos, nneg, a1b, a2b, a1u, a2u,
               xb, xu, rtb, rtu, dnb, dnu):
    j = pl.program_id(0)
    seg = (lax.broadcasted_iota(jnp.int32, (_SEG, _LANES), 1) // _DOUT
           == lax.broadcasted_iota(jnp.int32, (_SEG, _LANES), 0))
    expand = seg.astype(jnp.float32)

    @pl.when(j == 0)
    def _init():
        for hs, a1, a2, rt, dn, x in ((hsb, a1b, a2b, rtb, dnb, xb),
                                      (hsu, a1u, a2u, rtu, dnu, xu)):
            h = hs[...]
            r = jnp.dot(h, a1[...], preferred_element_type=jnp.float32)
            e = jnp.exp(-_leaky(
                r + jnp.dot(h, a2[...], preferred_element_type=jnp.float32)))
            rt[...] = r
            dn[...] = e
            x[...] = jnp.dot(e, expand,
                             preferred_element_type=jnp.float32) * h

    for nb, a2, rt, dn, x in ((npos, a2b, rtb, dnb, xb),
                              (nneg, a2u, rtu, dnu, xu)):
        h = nb[...]
        e = jnp.exp(-_leaky(
            rt[...] + jnp.dot(h, a2[...],
                              preferred_element_type=jnp.float32)))
        dn[...] += e
        x[...] += jnp.dot(e, expand, preferred_element_type=jnp.float32) * h

    @pl.when(j == _DEG - 1)
    def _final():
        for x, dn in ((xb, dnb), (xu, dnu)):
            d = jnp.dot(dn[...] + 1e-16, expand,
                        preferred_element_type=jnp.float32)
            v = x[...] / d
            x[...] = jnp.where(v > 0, v, jnp.exp(v) - 1.0)


def _attention(hsb, hsu, npos, nneg, a1b, a2b, a1u, a2u):
    return pl.pallas_call(
        _attn_body,
        grid=(_DEG,),
        in_specs=[
            pl.BlockSpec((_TB, _LANES), lambda j: (0, 0)),
            pl.BlockSpec((_TB, _LANES), lambda j: (0, 0)),
            pl.BlockSpec((_TB, _LANES), lambda j: (j, 0)),
            pl.BlockSpec((_TB, _LANES), lambda j: (j, 0)),
            pl.BlockSpec((_DIN, _SEG), lambda j: (0, 0)),
            pl.BlockSpec((_DIN, _SEG), lambda j: (0, 0)),
            pl.BlockSpec((_DIN, _SEG), lambda j: (0, 0)),
            pl.BlockSpec((_DIN, _SEG), lambda j: (0, 0)),
        ],
        out_specs=[
            pl.BlockSpec((_TB, _LANES), lambda j: (0, 0)),
            pl.BlockSpec((_TB, _LANES), lambda j: (0, 0)),
        ],
        out_shape=[
            jax.ShapeDtypeStruct((_TB, _LANES), jnp.float32),
            jax.ShapeDtypeStruct((_TB, _LANES), jnp.float32),
        ],
        scratch_shapes=[pltpu.VMEM((_TB, _SEG), jnp.float32)] * 4,
    )(hsb, hsu, npos, nneg, a1b, a2b, a1u, a2u)


def kernel(nodes, neigh_pos, neigh_neg, feat_table,
           W_bal, a_bal, W_unbal, a_unbal):
    hcat = _project(feat_table, W_bal, W_unbal)
    tbl = hcat.reshape(4 * _N, _DOUT)

    zs = jnp.zeros(_SELF_PAD - _B, jnp.int32)
    zr = jnp.zeros((_BPAD - _B, _DEG), jnp.int32)
    isb = jnp.concatenate([nodes * 4, zs])
    isu = isb + 1
    ipos = jnp.concatenate([neigh_pos, zr], axis=0)
    ineg = jnp.concatenate([neigh_neg, zr], axis=0)
    ipt, int_ = _transpose_idx(ipos, ineg)

    sb, su, nbp, nbn = _gather(tbl, isb, isu, ipt, int_)

    hs2b = sb.reshape(-1, _LANES)[:_TB]
    hs2u = su.reshape(-1, _LANES)[:_TB]
    np2 = nbp.reshape(-1, _LANES)
    nn2 = nbn.reshape(-1, _LANES)

    eye = jnp.eye(_SEG, dtype=jnp.float32)
    a1b = jnp.kron(eye, a_bal[0, :_DOUT][:, None])
    a2b = jnp.kron(eye, a_bal[0, _DOUT:][:, None])
    a1u = jnp.kron(eye, a_unbal[0, :_DOUT][:, None])
    a2u = jnp.kron(eye, a_unbal[0, _DOUT:][:, None])

    xb2, xu2 = _attention(hs2b, hs2u, np2, nn2, a1b, a2b, a1u, a2u)
    return (xb2.reshape(-1, _DOUT)[:_B], xu2.reshape(-1, _DOUT)[:_B])


# speedup vs baseline: 1.4554x; 1.3055x over previous
import jax
import jax.numpy as jnp
from jax import lax
from jax.experimental import pallas as pl
from jax.experimental.pallas import tpu as pltpu
from jax.experimental.pallas import tpu_sc as plsc

_N = 100000
_DIN = 128
_DOUT = 32
_B = 10000
_DEG = 16
_ALPHA = 0.2

_NC, _NS = 2, 16
_NW = _NC * _NS
_LANES = 128

_BPAD = 10240

_SELF_IR = 5
_SELF_PW = _SELF_IR * _LANES
_SELF_PAD = 2 * _BPAD
_NB_IR = 40
_NB_PW = _NB_IR * _LANES
_NB_PAD = _NW * _NB_PW
_CH_IR = 10
_CHUNK = _CH_IR * _LANES


def _leaky(x):
    return jnp.where(x >= 0, x, _ALPHA * x)


_PROJ_ROWS = 2000


def _proj_body(f_ref, wb_ref, wu_ref, h_ref):
    f = f_ref[...]
    hb = jnp.dot(f, wb_ref[...], preferred_element_type=jnp.float32)
    hu = jnp.dot(f, wu_ref[...], preferred_element_type=jnp.float32)
    z = jnp.zeros((_PROJ_ROWS, _DIN - 2 * _DOUT), jnp.float32)
    h_ref[...] = jnp.concatenate([hb, hu, z], axis=1)


def _project(feat, wb, wu):
    return pl.pallas_call(
        _proj_body,
        grid=(_N // _PROJ_ROWS,),
        in_specs=[
            pl.BlockSpec((_PROJ_ROWS, _DIN), lambda i: (i, 0)),
            pl.BlockSpec((_DIN, _DOUT), lambda i: (0, 0)),
            pl.BlockSpec((_DIN, _DOUT), lambda i: (0, 0)),
        ],
        out_specs=pl.BlockSpec((_PROJ_ROWS, _DIN), lambda i: (i, 0)),
        out_shape=jax.ShapeDtypeStruct((_N, _DIN), jnp.float32),
    )(feat, wb, wu)


def _tr_body(p_ref, n_ref, pt_ref, nt_ref):
    rows = lax.broadcasted_iota(jnp.int32, (_BPAD, _DEG), 0)
    p = jnp.where(rows < _B, p_ref[...], 0)
    n = jnp.where(rows < _B, n_ref[...], 0)
    pt_ref[...] = jnp.transpose(p) * 4
    nt_ref[...] = jnp.transpose(n) * 4 + 1


def _transpose_idx(ipos, ineg):
    return pl.pallas_call(
        _tr_body,
        in_specs=[pl.BlockSpec((_BPAD, _DEG), lambda i: (0, 0))] * 2,
        out_specs=[pl.BlockSpec((_DEG, _BPAD), lambda i: (0, 0))] * 2,
        out_shape=[jax.ShapeDtypeStruct((_DEG, _BPAD), jnp.int32)] * 2,
        grid=(1,),
    )(ipos, ineg)


def _fire(tbl, idx, i_off, rows, sem, n_ops=_CH_IR):
    return [
        pltpu.async_copy(tbl.at[idx.at[pl.ds(i_off + j * _LANES, _LANES)]],
                         rows.at[pl.ds(j * _LANES, _LANES)], sem)
        for j in range(n_ops)
    ]


def _gather_body(tbl, isel, ipos, ineg,
                 ss, nbp, nbn,
                 idx_s, idx_n, rows_a, rows_b, sem_a, sem_b):
    wid = lax.axis_index("s") * _NC + lax.axis_index("c")
    s_base = pl.multiple_of(wid * _SELF_PW, _SELF_PW)
    n_base = pl.multiple_of(wid * _NB_PW, _NB_PW)
    j_col = wid // _NC
    r_base = pl.multiple_of((wid % _NC) * _NB_PW, _NB_PW)

    pltpu.sync_copy(isel.at[pl.ds(s_base, _SELF_PW)], idx_s)
    ds = _fire(tbl, idx_s, 0, rows_a, sem_a, _SELF_IR)
    for d in ds:
        d.wait()
    pltpu.sync_copy(rows_a.at[pl.ds(0, _SELF_PW)],
                    ss.at[pl.ds(s_base, _SELF_PW)])

    for itab, out in ((ipos, nbp), (ineg, nbn)):
        pltpu.sync_copy(itab.at[j_col, pl.ds(r_base, _NB_PW)], idx_n)

        def pair(p, carry):
            off_a = pl.multiple_of(p * (2 * _CHUNK), 2 * _CHUNK)
            off_b = off_a + _CHUNK
            da = _fire(tbl, idx_n, off_a, rows_a, sem_a)
            db = _fire(tbl, idx_n, off_b, rows_b, sem_b)
            for d in da:
                d.wait()
            pltpu.sync_copy(rows_a, out.at[pl.ds(n_base + off_a, _CHUNK)])
            for d in db:
                d.wait()
            pltpu.sync_copy(rows_b, out.at[pl.ds(n_base + off_b, _CHUNK)])
            return carry

        lax.fori_loop(0, _NB_PW // (2 * _CHUNK), pair, 0)


def _gather(tbl, isel, ipos, ineg):
    mesh = plsc.VectorSubcoreMesh(core_axis_name="c", subcore_axis_name="s")
    f = pl.kernel(
        _gather_body,
        mesh=mesh,
        out_type=[
            jax.ShapeDtypeStruct((_SELF_PAD, _DOUT), jnp.float32),
            jax.ShapeDtypeStruct((_NB_PAD, _DOUT), jnp.float32),
            jax.ShapeDtypeStruct((_NB_PAD, _DOUT), jnp.float32),
        ],
        scratch_types=[
            pltpu.VMEM((_SELF_PW,), jnp.int32),
            pltpu.VMEM((_NB_PW,), jnp.int32),
            pltpu.VMEM((_CHUNK, _DOUT), jnp.float32),
            pltpu.VMEM((_CHUNK, _DOUT), jnp.float32),
            pltpu.SemaphoreType.DMA,
            pltpu.SemaphoreType.DMA,
        ],
        compiler_params=pltpu.CompilerParams(use_tc_tiling_on_sc=False,
                                             needs_layout_passes=False),
    )
    return f(tbl, isel, ipos, ineg)


_SEG = _LANES // _DOUT
_TB = _BPAD // _SEG
_TBV = _B // _SEG


def _attn_body(hss, npos, nneg, a1b, a2b, a1u, a2u,
               xb, xu, rtb, rtu, dnb, dnu):
    j = pl.program_id(0)
    seg = (lax.broadcasted_iota(jnp.int32, (_SEG, _LANES), 1) // _DOUT
           == lax.broadcasted_iota(jnp.int32, (_SEG, _LANES), 0))
    expand = seg.astype(jnp.float32)

    @pl.when(j == 0)
    def _init():
        for head, (a1, a2, rt, dn, x) in enumerate(
                ((a1b, a2b, rtb, dnb, xb), (a1u, a2u, rtu, dnu, xu))):
            h = hss[head][...]
            r = jnp.dot(h, a1[...], preferred_element_type=jnp.float32)
            e = jnp.exp(-_leaky(
                r + jnp.dot(h, a2[...], preferred_element_type=jnp.float32)))
            rt[...] = r
            dn[...] = e
            x[...] = (jnp.dot(e, expand,
                              preferred_element_type=jnp.float32) * h)[:_TBV]

    for nb, a2, rt, dn, x in ((npos, a2b, rtb, dnb, xb),
                              (nneg, a2u, rtu, dnu, xu)):
        h = nb[...]
        e = jnp.exp(-_leaky(
            rt[...] + jnp.dot(h, a2[...],
                              preferred_element_type=jnp.float32)))
        dn[...] += e
        x[...] += (jnp.dot(e, expand,
                           preferred_element_type=jnp.float32) * h)[:_TBV]

    @pl.when(j == _DEG - 1)
    def _final():
        for x, dn in ((xb, dnb), (xu, dnu)):
            d = jnp.dot(dn[...] + 1e-16, expand,
                        preferred_element_type=jnp.float32)
            v = x[...] / d[:_TBV]
            x[...] = jnp.where(v > 0, v, jnp.exp(v) - 1.0)


def _attention(hss2, npos, nneg, a1b, a2b, a1u, a2u):
    def hs_spec(head):
        return pl.BlockSpec((_TB, _LANES), lambda j, h=head: (h, 0))

    body = (lambda hs0, hs1, *rest:
            _attn_body((hs0, hs1), *rest))
    return pl.pallas_call(
        body,
        grid=(_DEG,),
        in_specs=[
            hs_spec(0),
            hs_spec(1),
            pl.BlockSpec((_TB, _LANES), lambda j: (j, 0)),
            pl.BlockSpec((_TB, _LANES), lambda j: (j, 0)),
            pl.BlockSpec((_DIN, _SEG), lambda j: (0, 0)),
            pl.BlockSpec((_DIN, _SEG), lambda j: (0, 0)),
            pl.BlockSpec((_DIN, _SEG), lambda j: (0, 0)),
            pl.BlockSpec((_DIN, _SEG), lambda j: (0, 0)),
        ],
        out_specs=[
            pl.BlockSpec((_TBV, _LANES), lambda j: (0, 0)),
            pl.BlockSpec((_TBV, _LANES), lambda j: (0, 0)),
        ],
        out_shape=[
            jax.ShapeDtypeStruct((_TBV, _LANES), jnp.float32),
            jax.ShapeDtypeStruct((_TBV, _LANES), jnp.float32),
        ],
        scratch_shapes=[pltpu.VMEM((_TB, _SEG), jnp.float32),
                        pltpu.VMEM((_TB, _SEG), jnp.float32),
                        pltpu.VMEM((_TB, _SEG), jnp.float32),
                        pltpu.VMEM((_TB, _SEG), jnp.float32)],
    )(hss2, hss2, npos, nneg, a1b, a2b, a1u, a2u)


def kernel(nodes, neigh_pos, neigh_neg, feat_table,
           W_bal, a_bal, W_unbal, a_unbal):
    hcat = _project(feat_table, W_bal, W_unbal)
    tbl = hcat.reshape(4 * _N, _DOUT)

    zs = jnp.zeros(_BPAD - _B, jnp.int32)
    isb = jnp.concatenate([nodes * 4, zs])
    isel = jnp.concatenate([isb, isb + 1])
    ipt, int_ = _transpose_idx(neigh_pos, neigh_neg)

    ss, nbp, nbn = _gather(tbl, isel, ipt, int_)

    hss2 = ss.reshape(-1, _LANES)
    np2 = nbp.reshape(-1, _LANES)
    nn2 = nbn.reshape(-1, _LANES)

    eye = jnp.eye(_SEG, dtype=jnp.float32)
    a1b = jnp.kron(eye, a_bal[0, :_DOUT][:, None])
    a2b = jnp.kron(eye, a_bal[0, _DOUT:][:, None])
    a1u = jnp.kron(eye, a_unbal[0, :_DOUT][:, None])
    a2u = jnp.kron(eye, a_unbal[0, _DOUT:][:, None])

    xb2, xu2 = _attention(hss2, np2, nn2, a1b, a2b, a1u, a2u)
    return (xb2.reshape(_B, _DOUT), xu2.reshape(_B, _DOUT))


# speedup vs baseline: 1.4928x; 1.0257x over previous
import jax
import jax.numpy as jnp
from jax import lax
from jax.experimental import pallas as pl
from jax.experimental.pallas import tpu as pltpu
from jax.experimental.pallas import tpu_sc as plsc

_N = 100000
_DIN = 128
_DOUT = 32
_B = 10000
_DEG = 16
_ALPHA = 0.2

_NC, _NS = 2, 16
_NW = _NC * _NS
_LANES = 128

_BPAD = 10240

_SELF_IR = 5
_SELF_PW = _SELF_IR * _LANES
_SELF_PAD = 2 * _BPAD
_NB_IR = 40
_NB_PW = _NB_IR * _LANES
_NB_PAD = _NW * _NB_PW
_CH_IR = 10
_CHUNK = _CH_IR * _LANES


def _leaky(x):
    return jnp.where(x >= 0, x, _ALPHA * x)


_PROJ_ROWS = 2000


def _proj_body(f_ref, wb_ref, wu_ref, h_ref):
    f = f_ref[...]
    hb = jnp.dot(f, wb_ref[...], preferred_element_type=jnp.float32)
    hu = jnp.dot(f, wu_ref[...], preferred_element_type=jnp.float32)
    z = jnp.zeros((_PROJ_ROWS, _DIN - 2 * _DOUT), jnp.float32)
    h_ref[...] = jnp.concatenate([hb, hu, z], axis=1)


def _project(feat, wb, wu):
    return pl.pallas_call(
        _proj_body,
        grid=(_N // _PROJ_ROWS,),
        in_specs=[
            pl.BlockSpec((_PROJ_ROWS, _DIN), lambda i: (i, 0)),
            pl.BlockSpec((_DIN, _DOUT), lambda i: (0, 0)),
            pl.BlockSpec((_DIN, _DOUT), lambda i: (0, 0)),
        ],
        out_specs=pl.BlockSpec((_PROJ_ROWS, _DIN), lambda i: (i, 0)),
        out_shape=jax.ShapeDtypeStruct((_N, _DIN), jnp.float32),
    )(feat, wb, wu)


def _tr_body(p_ref, n_ref, pt_ref, nt_ref):
    rows = lax.broadcasted_iota(jnp.int32, (_BPAD, _DEG), 0)
    p = jnp.where(rows < _B, p_ref[...], 0)
    n = jnp.where(rows < _B, n_ref[...], 0)
    pt_ref[...] = jnp.transpose(p) * 4
    nt_ref[...] = jnp.transpose(n) * 4 + 1


def _transpose_idx(ipos, ineg):
    return pl.pallas_call(
        _tr_body,
        in_specs=[pl.BlockSpec((_BPAD, _DEG), lambda i: (0, 0))] * 2,
        out_specs=[pl.BlockSpec((_DEG, _BPAD), lambda i: (0, 0))] * 2,
        out_shape=[jax.ShapeDtypeStruct((_DEG, _BPAD), jnp.int32)] * 2,
        grid=(1,),
    )(ipos, ineg)


def _fire(tbl, idx, i_off, rows, sem, n_ops=_CH_IR):
    return [
        pltpu.async_copy(tbl.at[idx.at[pl.ds(i_off + j * _LANES, _LANES)]],
                         rows.at[pl.ds(j * _LANES, _LANES)], sem)
        for j in range(n_ops)
    ]


def _gather_body(tbl, isel, ipos, ineg,
                 ss, nbp, nbn,
                 idx_s, idx_n, rows_a, rows_b, sem_a, sem_b):
    wid = lax.axis_index("c") * _NS + lax.axis_index("s")
    s_base = pl.multiple_of(wid * _SELF_PW, _SELF_PW)
    n_base = pl.multiple_of(wid * _NB_PW, _NB_PW)
    j_col = wid // _NC
    r_base = pl.multiple_of((wid % _NC) * _NB_PW, _NB_PW)

    pltpu.sync_copy(isel.at[pl.ds(s_base, _SELF_PW)], idx_s)
    ds = _fire(tbl, idx_s, 0, rows_a, sem_a, _SELF_IR)
    for d in ds:
        d.wait()
    pltpu.sync_copy(rows_a.at[pl.ds(0, _SELF_PW)],
                    ss.at[pl.ds(s_base, _SELF_PW)])

    for itab, out in ((ipos, nbp), (ineg, nbn)):
        pltpu.sync_copy(itab.at[j_col, pl.ds(r_base, _NB_PW)], idx_n)

        def pair(p, carry):
            off_a = pl.multiple_of(p * (2 * _CHUNK), 2 * _CHUNK)
            off_b = off_a + _CHUNK
            da = _fire(tbl, idx_n, off_a, rows_a, sem_a)
            db = _fire(tbl, idx_n, off_b, rows_b, sem_b)
            for d in da:
                d.wait()
            pltpu.sync_copy(rows_a, out.at[pl.ds(n_base + off_a, _CHUNK)])
            for d in db:
                d.wait()
            pltpu.sync_copy(rows_b, out.at[pl.ds(n_base + off_b, _CHUNK)])
            return carry

        lax.fori_loop(0, _NB_PW // (2 * _CHUNK), pair, 0)


def _gather(tbl, isel, ipos, ineg):
    mesh = plsc.VectorSubcoreMesh(core_axis_name="c", subcore_axis_name="s")
    f = pl.kernel(
        _gather_body,
        mesh=mesh,
        out_type=[
            jax.ShapeDtypeStruct((_SELF_PAD, _DOUT), jnp.float32),
            jax.ShapeDtypeStruct((_NB_PAD, _DOUT), jnp.float32),
            jax.ShapeDtypeStruct((_NB_PAD, _DOUT), jnp.float32),
        ],
        scratch_types=[
            pltpu.VMEM((_SELF_PW,), jnp.int32),
            pltpu.VMEM((_NB_PW,), jnp.int32),
            pltpu.VMEM((_CHUNK, _DOUT), jnp.float32),
            pltpu.VMEM((_CHUNK, _DOUT), jnp.float32),
            pltpu.SemaphoreType.DMA,
            pltpu.SemaphoreType.DMA,
        ],
        compiler_params=pltpu.CompilerParams(use_tc_tiling_on_sc=False,
                                             needs_layout_passes=False),
    )
    return f(tbl, isel, ipos, ineg)


_SEG = _LANES // _DOUT
_TB = _BPAD // _SEG
_TBV = _B // _SEG


def _attn_body(hss, npos, nneg, a1b, a2b, a1u, a2u,
               xb, xu, rtb, rtu, dnb, dnu):
    j = pl.program_id(0)
    seg = (lax.broadcasted_iota(jnp.int32, (_SEG, _LANES), 1) // _DOUT
           == lax.broadcasted_iota(jnp.int32, (_SEG, _LANES), 0))
    expand = seg.astype(jnp.float32)

    @pl.when(j == 0)
    def _init():
        for head, (a1, a2, rt, dn, x) in enumerate(
                ((a1b, a2b, rtb, dnb, xb), (a1u, a2u, rtu, dnu, xu))):
            h = hss[head][...]
            r = jnp.dot(h, a1[...], preferred_element_type=jnp.float32)
            e = jnp.exp(-_leaky(
                r + jnp.dot(h, a2[...], preferred_element_type=jnp.float32)))
            rt[...] = r
            dn[...] = e
            x[...] = (jnp.dot(e, expand,
                              preferred_element_type=jnp.float32) * h)[:_TBV]

    for nb, a2, rt, dn, x in ((npos, a2b, rtb, dnb, xb),
                              (nneg, a2u, rtu, dnu, xu)):
        h = nb[...]
        e = jnp.exp(-_leaky(
            rt[...] + jnp.dot(h, a2[...],
                              preferred_element_type=jnp.float32)))
        dn[...] += e
        x[...] += (jnp.dot(e, expand,
                           preferred_element_type=jnp.float32) * h)[:_TBV]

    @pl.when(j == _DEG - 1)
    def _final():
        for x, dn in ((xb, dnb), (xu, dnu)):
            d = jnp.dot(dn[...] + 1e-16, expand,
                        preferred_element_type=jnp.float32)
            v = x[...] / d[:_TBV]
            x[...] = jnp.where(v > 0, v, jnp.exp(v) - 1.0)


def _attention(hss2, npos, nneg, a1b, a2b, a1u, a2u):
    def hs_spec(head):
        return pl.BlockSpec((_TB, _LANES), lambda j, h=head: (h, 0))

    body = (lambda hs0, hs1, *rest:
            _attn_body((hs0, hs1), *rest))
    return pl.pallas_call(
        body,
        grid=(_DEG,),
        in_specs=[
            hs_spec(0),
            hs_spec(1),
            pl.BlockSpec((_TB, _LANES), lambda j: (j, 0)),
            pl.BlockSpec((_TB, _LANES), lambda j: (j, 0)),
            pl.BlockSpec((_DIN, _SEG), lambda j: (0, 0)),
            pl.BlockSpec((_DIN, _SEG), lambda j: (0, 0)),
            pl.BlockSpec((_DIN, _SEG), lambda j: (0, 0)),
            pl.BlockSpec((_DIN, _SEG), lambda j: (0, 0)),
        ],
        out_specs=[
            pl.BlockSpec((_TBV, _LANES), lambda j: (0, 0)),
            pl.BlockSpec((_TBV, _LANES), lambda j: (0, 0)),
        ],
        out_shape=[
            jax.ShapeDtypeStruct((_TBV, _LANES), jnp.float32),
            jax.ShapeDtypeStruct((_TBV, _LANES), jnp.float32),
        ],
        scratch_shapes=[pltpu.VMEM((_TB, _SEG), jnp.float32),
                        pltpu.VMEM((_TB, _SEG), jnp.float32),
                        pltpu.VMEM((_TB, _SEG), jnp.float32),
                        pltpu.VMEM((_TB, _SEG), jnp.float32)],
    )(hss2, hss2, npos, nneg, a1b, a2b, a1u, a2u)


def kernel(nodes, neigh_pos, neigh_neg, feat_table,
           W_bal, a_bal, W_unbal, a_unbal):
    hcat = _project(feat_table, W_bal, W_unbal)
    tbl = hcat.reshape(4 * _N, _DOUT)

    zs = jnp.zeros(_BPAD - _B, jnp.int32)
    isb = jnp.concatenate([nodes * 4, zs])
    isel = jnp.concatenate([isb, isb + 1])
    ipt, int_ = _transpose_idx(neigh_pos, neigh_neg)

    ss, nbp, nbn = _gather(tbl, isel, ipt, int_)

    hss2 = ss.reshape(-1, _LANES)
    np2 = nbp.reshape(-1, _LANES)
    nn2 = nbn.reshape(-1, _LANES)

    eye = jnp.eye(_SEG, dtype=jnp.float32)
    a1b = jnp.kron(eye, a_bal[0, :_DOUT][:, None])
    a2b = jnp.kron(eye, a_bal[0, _DOUT:][:, None])
    a1u = jnp.kron(eye, a_unbal[0, :_DOUT][:, None])
    a2u = jnp.kron(eye, a_unbal[0, _DOUT:][:, None])

    xb2, xu2 = _attention(hss2, np2, nn2, a1b, a2b, a1u, a2u)
    return (xb2.reshape(_B, _DOUT), xu2.reshape(_B, _DOUT))
